# baseline (speedup 1.0000x reference)
import jax
import jax.numpy as jnp
from jax import lax
from jax.experimental import pallas as pl
from jax.experimental.pallas import tpu as pltpu
from jax.experimental.pallas import tpu_sc as plsc

E = 160000
D = 256
DW = D // 2
NC = 2
NS = 16
NW = NC * NS
PER_W = E // NW
C = 96
NFULL = PER_W // C
TAIL = PER_W - NFULL * C
LOOP_CHUNKS = NFULL - 2
G = C // 16
OV_LEN = NFULL * C + 16


def _dot_group(ub, mb, row0):
    lane = lax.iota(jnp.int32, 16)
    zero = jnp.zeros((16,), jnp.float32)

    def jbody(j, res):
        e = row0 + j
        a0 = zero
        a1 = zero
        for s in range(DW // 16):
            u = plsc.bitcast(ub[e, pl.ds(s * 16, 16)], jnp.bfloat16)
            m = plsc.bitcast(mb[e, pl.ds(s * 16, 16)], jnp.bfloat16)
            pa, pb = plsc.unpack(u * m, format=plsc.PackFormat.INTERLEAVED)
            a0 = a0 + pa
            a1 = a1 + pb
        acc = a0 + a1
        for k in (1, 2, 4, 8):
            acc = acc + jnp.take_along_axis(acc, lane ^ k, axis=0,
                                            mode="promise_in_bounds")
        return jnp.where(lane == j, acc, res)

    return lax.fori_loop(0, 16, jbody, zero)


def _compute_chunk(ub, mb, ov, out_off):
    def body(g, carry):
        ov[pl.ds(out_off + g * 16, 16)] = _dot_group(ub, mb, g * 16)
        return carry

    lax.fori_loop(0, G, body, 0)


def _sc_body(xu, xm, eli, out,
             iu0, iu1, im0, im1, iut, imt,
             ub0, ub1, mb0, mb1, ov,
             siu0, siu1, sim0, sim1, su0, su1, sm0, sm1):
    c = lax.axis_index("c")
    s = lax.axis_index("s")
    wid = s * NC + c
    base = wid * PER_W

    ius = (iu0, iu1)
    ims = (im0, im1)
    ubs = (ub0, ub1)
    mbs = (mb0, mb1)
    sius = (siu0, siu1)
    sims = (sim0, sim1)
    sus = (su0, su1)
    sms = (sm0, sm1)

    def fire_idx(k, b):
        off = base + k * C
        pltpu.async_copy(eli.at[pl.ds(off, C)], ius[b], sius[b])
        pltpu.async_copy(eli.at[pl.ds(E + off, C)], ims[b], sims[b])

    def wait_idx(k, b):
        off = base + k * C
        pltpu.make_async_copy(eli.at[pl.ds(off, C)], ius[b], sius[b]).wait()
        pltpu.make_async_copy(eli.at[pl.ds(E + off, C)], ims[b], sims[b]).wait()

    def fire_rows(b):
        pltpu.async_copy(xu.at[ius[b]], ubs[b], sus[b])
        pltpu.async_copy(xm.at[ims[b]], mbs[b], sms[b])

    def wait_rows(b):
        pltpu.make_async_copy(xu.at[ius[b]], ubs[b], sus[b]).wait()
        pltpu.make_async_copy(xm.at[ims[b]], mbs[b], sms[b]).wait()

    fire_idx(0, 0)
    wait_idx(0, 0)
    fire_rows(0)
    fire_idx(1, 1)

    def pair_body(p, carry):
        for half in range(2):
            k = 2 * p + half
            b = half
            nb = 1 - half
            wait_idx(k + 1, nb)
            fire_rows(nb)
            wait_rows(b)
            fire_idx(k + 2, b)
            _compute_chunk(ubs[b], mbs[b], ov, k * C)
        return carry

    lax.fori_loop(0, LOOP_CHUNKS // 2, pair_body, 0)

    k50 = LOOP_CHUNKS
    k51 = LOOP_CHUNKS + 1
    wait_idx(k51, 1)
    fire_rows(1)
    wait_rows(0)
    _compute_chunk(ub0, mb0, ov, k50 * C)

    toff = base + NFULL * C
    pltpu.sync_copy(eli.at[pl.ds(toff, TAIL)], iut)
    pltpu.sync_copy(eli.at[pl.ds(E + toff, TAIL)], imt)
    pltpu.async_copy(xu.at[iut], ub0.at[pl.ds(0, TAIL)], su0)
    pltpu.async_copy(xm.at[imt], mb0.at[pl.ds(0, TAIL)], sm0)

    wait_rows(1)
    _compute_chunk(ub1, mb1, ov, k51 * C)

    pltpu.make_async_copy(xu.at[iut], ub0.at[pl.ds(0, TAIL)], su0).wait()
    pltpu.make_async_copy(xm.at[imt], mb0.at[pl.ds(0, TAIL)], sm0).wait()
    ov[pl.ds(NFULL * C, 16)] = _dot_group(ub0, mb0, 0)

    pltpu.sync_copy(ov.at[pl.ds(0, PER_W)], out.at[pl.ds(base, PER_W)])


def kernel(x_user, x_movie, edge_label_index):
    mesh = plsc.VectorSubcoreMesh(core_axis_name="c", subcore_axis_name="s")
    run = pl.kernel(
        _sc_body,
        out_type=jax.ShapeDtypeStruct((E,), jnp.float32),
        mesh=mesh,
        compiler_params=pltpu.CompilerParams(
            use_tc_tiling_on_sc=False, needs_layout_passes=False),
        scratch_types=[
            pltpu.VMEM((C,), jnp.int32),
            pltpu.VMEM((C,), jnp.int32),
            pltpu.VMEM((C,), jnp.int32),
            pltpu.VMEM((C,), jnp.int32),
            pltpu.VMEM((TAIL,), jnp.int32),
            pltpu.VMEM((TAIL,), jnp.int32),
            pltpu.VMEM((C, DW), jnp.int32),
            pltpu.VMEM((C, DW), jnp.int32),
            pltpu.VMEM((C, DW), jnp.int32),
            pltpu.VMEM((C, DW), jnp.int32),
            pltpu.VMEM((OV_LEN,), jnp.float32),
            pltpu.SemaphoreType.DMA,
            pltpu.SemaphoreType.DMA,
            pltpu.SemaphoreType.DMA,
            pltpu.SemaphoreType.DMA,
            pltpu.SemaphoreType.DMA,
            pltpu.SemaphoreType.DMA,
            pltpu.SemaphoreType.DMA,
            pltpu.SemaphoreType.DMA,
        ],
    )
    def to_packed(t):
        u = jax.lax.bitcast_convert_type(t, jnp.uint32)
        r = (u + 0x7FFF + ((u >> 16) & 1)) >> 16
        packed = r[:, 0::2] | (r[:, 1::2] << 16)
        return jax.lax.bitcast_convert_type(packed, jnp.int32)

    return run(to_packed(x_user), to_packed(x_movie),
               edge_label_index.reshape(-1))


# speedup vs baseline: 7.5560x; 7.5560x over previous
import jax
import jax.numpy as jnp
from jax import lax
from jax.experimental import pallas as pl
from jax.experimental.pallas import tpu as pltpu
from jax.experimental.pallas import tpu_sc as plsc

E = 160000
D = 256
DW = D // 2
NC = 2
NS = 16
NW = NC * NS
PER_W = E // NW
C = 96
NFULL = PER_W // C
TAIL = PER_W - NFULL * C
LOOP_CHUNKS = NFULL - 2
G = C // 16
OV_LEN = NFULL * C + 16


def _dot_group(ub, mb, row0):
    lane = lax.iota(jnp.int32, 16)
    zero = jnp.zeros((16,), jnp.float32)

    def jbody(j, res):
        e = row0 + j
        a0 = zero
        a1 = zero
        for s in range(DW // 16):
            u = plsc.bitcast(ub[e, pl.ds(s * 16, 16)], jnp.bfloat16)
            m = plsc.bitcast(mb[e, pl.ds(s * 16, 16)], jnp.bfloat16)
            pa, pb = plsc.unpack(u * m, format=plsc.PackFormat.INTERLEAVED)
            a0 = a0 + pa
            a1 = a1 + pb
        acc = a0 + a1
        for k in (1, 2, 4, 8):
            acc = acc + jnp.take_along_axis(acc, lane ^ k, axis=0,
                                            mode="promise_in_bounds")
        return jnp.where(lane == j, acc, res)

    return lax.fori_loop(0, 16, jbody, zero)


def _compute_chunk(ub, mb, ov, out_off):
    def body(g, carry):
        ov[pl.ds(out_off + g * 16, 16)] = _dot_group(ub, mb, g * 16)
        return carry

    lax.fori_loop(0, G, body, 0)


def _sc_body(xu, xm, eli, out,
             iu0, iu1, im0, im1, iut, imt,
             ub0, ub1, mb0, mb1, ov,
             siu0, siu1, sim0, sim1, su0, su1, sm0, sm1):
    c = lax.axis_index("c")
    s = lax.axis_index("s")
    wid = s * NC + c
    base = wid * PER_W

    ius = (iu0, iu1)
    ims = (im0, im1)
    ubs = (ub0, ub1)
    mbs = (mb0, mb1)
    sius = (siu0, siu1)
    sims = (sim0, sim1)
    sus = (su0, su1)
    sms = (sm0, sm1)

    def fire_idx(k, b):
        off = base + k * C
        pltpu.async_copy(eli.at[pl.ds(off, C)], ius[b], sius[b])
        pltpu.async_copy(eli.at[pl.ds(E + off, C)], ims[b], sims[b])

    def wait_idx(k, b):
        off = base + k * C
        pltpu.make_async_copy(eli.at[pl.ds(off, C)], ius[b], sius[b]).wait()
        pltpu.make_async_copy(eli.at[pl.ds(E + off, C)], ims[b], sims[b]).wait()

    def fire_rows(b):
        pltpu.async_copy(xu.at[ius[b]], ubs[b], sus[b])
        pltpu.async_copy(xm.at[ims[b]], mbs[b], sms[b])

    def wait_rows(b):
        pltpu.make_async_copy(xu.at[ius[b]], ubs[b], sus[b]).wait()
        pltpu.make_async_copy(xm.at[ims[b]], mbs[b], sms[b]).wait()

    fire_idx(0, 0)
    wait_idx(0, 0)
    fire_rows(0)
    fire_idx(1, 1)

    def pair_body(p, carry):
        for half in range(2):
            k = 2 * p + half
            b = half
            nb = 1 - half
            wait_idx(k + 1, nb)
            fire_rows(nb)
            wait_rows(b)
            fire_idx(k + 2, b)
            _compute_chunk(ubs[b], mbs[b], ov, k * C)
        return carry

    lax.fori_loop(0, LOOP_CHUNKS // 2, pair_body, 0)

    k50 = LOOP_CHUNKS
    k51 = LOOP_CHUNKS + 1
    wait_idx(k51, 1)
    fire_rows(1)
    wait_rows(0)
    _compute_chunk(ub0, mb0, ov, k50 * C)

    toff = base + NFULL * C
    pltpu.sync_copy(eli.at[pl.ds(toff, TAIL)], iut)
    pltpu.sync_copy(eli.at[pl.ds(E + toff, TAIL)], imt)
    pltpu.async_copy(xu.at[iut], ub0.at[pl.ds(0, TAIL)], su0)
    pltpu.async_copy(xm.at[imt], mb0.at[pl.ds(0, TAIL)], sm0)

    wait_rows(1)
    _compute_chunk(ub1, mb1, ov, k51 * C)

    pltpu.make_async_copy(xu.at[iut], ub0.at[pl.ds(0, TAIL)], su0).wait()
    pltpu.make_async_copy(xm.at[imt], mb0.at[pl.ds(0, TAIL)], sm0).wait()
    ov[pl.ds(NFULL * C, 16)] = _dot_group(ub0, mb0, 0)

    pltpu.sync_copy(ov.at[pl.ds(0, PER_W)], out.at[pl.ds(base, PER_W)])


def kernel(x_user, x_movie, edge_label_index):
    mesh = plsc.VectorSubcoreMesh(core_axis_name="c", subcore_axis_name="s")
    run = pl.kernel(
        _sc_body,
        out_type=jax.ShapeDtypeStruct((E,), jnp.float32),
        mesh=mesh,
        compiler_params=pltpu.CompilerParams(
            use_tc_tiling_on_sc=False, needs_layout_passes=False),
        scratch_types=[
            pltpu.VMEM((C,), jnp.int32),
            pltpu.VMEM((C,), jnp.int32),
            pltpu.VMEM((C,), jnp.int32),
            pltpu.VMEM((C,), jnp.int32),
            pltpu.VMEM((TAIL,), jnp.int32),
            pltpu.VMEM((TAIL,), jnp.int32),
            pltpu.VMEM((C, DW), jnp.int32),
            pltpu.VMEM((C, DW), jnp.int32),
            pltpu.VMEM((C, DW), jnp.int32),
            pltpu.VMEM((C, DW), jnp.int32),
            pltpu.VMEM((OV_LEN,), jnp.float32),
            pltpu.SemaphoreType.DMA,
            pltpu.SemaphoreType.DMA,
            pltpu.SemaphoreType.DMA,
            pltpu.SemaphoreType.DMA,
            pltpu.SemaphoreType.DMA,
            pltpu.SemaphoreType.DMA,
            pltpu.SemaphoreType.DMA,
            pltpu.SemaphoreType.DMA,
        ],
    )
    def to_packed(t):
        u = jax.lax.bitcast_convert_type(t, jnp.uint32)
        r = (u + 0x7FFF + ((u >> 16) & 1)) >> 16
        packed = r[:, :DW] | (r[:, DW:] << 16)
        return jax.lax.bitcast_convert_type(packed, jnp.int32)

    return run(to_packed(x_user), to_packed(x_movie),
               edge_label_index.reshape(-1))
